# baseline (speedup 1.0000x reference)
import functools

import jax
import jax.numpy as jnp
from jax import lax
from jax.experimental import pallas as pl
from jax.experimental.pallas import tpu as pltpu
from jax.experimental.pallas import tpu_sc as plsc

_VOCAB = 1025
_HID = 768
_NTOK = 16384

_NC = 2
_NS = 16
_NW = _NC * _NS

_B_PER_W = _NTOK // _NW
_CHUNK = 64
_PAIRS = _B_PER_W // (2 * _CHUNK)


@functools.cache
def _build():
    mesh = plsc.VectorSubcoreMesh(core_axis_name="c", subcore_axis_name="s")

    @functools.partial(
        pl.kernel,
        mesh=mesh,
        out_type=jax.ShapeDtypeStruct((_NTOK, _HID), jnp.float32),
        scratch_types=[
            pltpu.VMEM((_B_PER_W,), jnp.int32),
            pltpu.VMEM((2, _CHUNK, _HID), jnp.float32),
            pltpu.SemaphoreType.DMA,
            pltpu.SemaphoreType.DMA,
            pltpu.SemaphoreType.DMA,
            pltpu.SemaphoreType.DMA,
        ],
    )
    def gather_kernel(table_hbm, idx_hbm, out_hbm, idx_v, rows_v,
                      gsem0, gsem1, ssem0, ssem1):
        wid = lax.axis_index("s") * _NC + lax.axis_index("c")
        base = wid * _B_PER_W
        pltpu.sync_copy(idx_hbm.at[pl.ds(base, _B_PER_W)], idx_v)

        def drain_scatter(buf, ssem):
            pltpu.make_async_copy(
                rows_v.at[buf], out_hbm.at[pl.ds(base, _CHUNK)], ssem).wait()

        def body(g, carry):
            @pl.when(g > 0)
            def _wait_prev():
                drain_scatter(0, ssem0)
                drain_scatter(1, ssem1)

            c0 = pl.multiple_of(2 * g * _CHUNK, 2 * _CHUNK)
            c1 = c0 + _CHUNK
            g0 = pltpu.async_copy(
                table_hbm.at[idx_v.at[pl.ds(c0, _CHUNK)]], rows_v.at[0], gsem0)
            g1 = pltpu.async_copy(
                table_hbm.at[idx_v.at[pl.ds(c1, _CHUNK)]], rows_v.at[1], gsem1)
            g0.wait()
            pltpu.async_copy(
                rows_v.at[0], out_hbm.at[pl.ds(base + c0, _CHUNK)], ssem0)
            g1.wait()
            pltpu.async_copy(
                rows_v.at[1], out_hbm.at[pl.ds(base + c1, _CHUNK)], ssem1)
            return carry

        lax.fori_loop(0, _PAIRS, body, 0)
        drain_scatter(0, ssem0)
        drain_scatter(1, ssem1)

    return gather_kernel


def kernel(input_ids, positions, embed_tokens):
    del positions
    return _build()(embed_tokens, input_ids.astype(jnp.int32))
